# baseline (speedup 1.0000x reference)
import functools

import jax
import jax.numpy as jnp
from jax import lax
from jax.experimental import pallas as pl
from jax.experimental.pallas import tpu as pltpu
from jax.experimental.pallas import tpu_sc as plsc

_L = 128
_D = 32
_NW = 32
_CR = 2
_CI = _CR * _L


def _emb_kernel(n_idx, table_hbm, idx_hbm, out_hbm,
                idx_v, q_v, g_a, g_b, o_a, o_b,
                sem_ga, sem_gb, sem_wa, sem_wb):
    wid = lax.axis_index("s") * 2 + lax.axis_index("c")
    ipw = n_idx // _NW
    base = wid * ipw
    pltpu.sync_copy(idx_hbm.at[pl.ds(base, ipw)], idx_v)

    iota = lax.iota(jnp.int32, 16)

    def qbody(i, carry):
        vec = idx_v[pl.ds(i * 16, 16)]
        q_v[pl.ds(i * 16, 16)] = vec >> 2
        return carry

    lax.fori_loop(0, ipw // 16, qbody, 0)

    n_chunks = ipw // _CI
    n_pairs = n_chunks // 2

    def g_start(g_buf, sem, c):
        for j in range(_CR):
            pltpu.async_copy(
                table_hbm.at[q_v.at[pl.ds(c * _CI + j * _L, _L)]],
                g_buf.at[pl.ds(j * _L, _L)], sem)

    def g_drain(g_buf, sem):
        pltpu.make_async_copy(table_hbm.at[pl.ds(0, _CI)], g_buf, sem).wait()

    def extract(c, g_buf, o_buf):
        def ebody(t, carry):
            vec = idx_v[pl.ds(c * _CI + t * 16, 16)]
            sub32 = (vec & 3) << 5
            p16 = iota + t * 16
            obase = iota * _D + t * (16 * _D)
            for w in range(_D):
                vals = plsc.load_gather(g_buf, [p16, sub32 + w])
                plsc.store_scatter(o_buf, [obase + w], vals)
            return carry

        lax.fori_loop(0, _CI // 16, ebody, 0)

    g_start(g_a, sem_ga, 0)
    g_start(g_b, sem_gb, 1)

    def body(p, carry):
        c0 = p * 2
        g_drain(g_a, sem_ga)
        extract(c0, g_a, o_a)
        wa = pltpu.async_copy(
            o_a, out_hbm.at[pl.ds((base + c0 * _CI) * _D, _CI * _D)], sem_wa)

        @pl.when(p < n_pairs - 1)
        def _():
            g_start(g_a, sem_ga, c0 + 2)

        g_drain(g_b, sem_gb)
        extract(c0 + 1, g_b, o_b)
        wb = pltpu.async_copy(
            o_b, out_hbm.at[pl.ds((base + (c0 + 1) * _CI) * _D, _CI * _D)],
            sem_wb)

        @pl.when(p < n_pairs - 1)
        def _():
            g_start(g_b, sem_gb, c0 + 3)

        wa.wait()
        wb.wait()
        return carry

    lax.fori_loop(0, n_pairs, body, 0)


def kernel(x, emb_weight):
    n, m = x.shape
    n_vocab = emb_weight.shape[0]
    n_idx = n * m
    idx1d = x.reshape(-1).astype(jnp.int32)
    table_v = emb_weight.reshape(n_vocab * _D // _L, _L)
    ipw = n_idx // _NW

    mesh = plsc.VectorSubcoreMesh(core_axis_name="c", subcore_axis_name="s")

    k = functools.partial(
        pl.kernel,
        mesh=mesh,
        out_type=jax.ShapeDtypeStruct((n_idx * _D,), jnp.float32),
        scratch_types=[
            pltpu.VMEM((ipw,), jnp.int32),
            pltpu.VMEM((ipw,), jnp.int32),
            pltpu.VMEM((_CI, _L), jnp.float32),
            pltpu.VMEM((_CI, _L), jnp.float32),
            pltpu.VMEM((_CI * _D,), jnp.float32),
            pltpu.VMEM((_CI * _D,), jnp.float32),
            pltpu.SemaphoreType.DMA,
            pltpu.SemaphoreType.DMA,
            pltpu.SemaphoreType.DMA,
            pltpu.SemaphoreType.DMA,
        ],
        compiler_params=pltpu.CompilerParams(needs_layout_passes=False),
    )(functools.partial(_emb_kernel, n_idx))

    out = k(table_v, idx1d)
    return out.reshape(n, m, _D)


# speedup vs baseline: 1.5920x; 1.5920x over previous
import functools

import jax
import jax.numpy as jnp
from jax import lax
from jax.experimental import pallas as pl
from jax.experimental.pallas import tpu as pltpu
from jax.experimental.pallas import tpu_sc as plsc

_D = 32
_NW = 32
_CR = 64


def _emb_kernel(n_rows, m, table_hbm, idx_hbm, out_hbm,
                idx_v, rows_a, rows_b, sem_ga, sem_gb, sem_wa, sem_wb):
    wid = lax.axis_index("s") * 2 + lax.axis_index("c")
    rpw = n_rows // _NW
    base = wid * rpw
    pltpu.sync_copy(idx_hbm.at[pl.ds(base, rpw)], idx_v)

    n_pairs = rpw // _CR // 2

    def g_start(buf, sem, c):
        def one(r, carry):
            pltpu.async_copy(
                table_hbm.at[idx_v.at[c * _CR + r]], buf.at[r], sem)
            return carry

        lax.fori_loop(0, _CR, one, 0)

    def g_drain(buf, sem):
        pltpu.make_async_copy(out_hbm.at[pl.ds(0, _CR)], buf, sem).wait()

    g_start(rows_a, sem_ga, 0)
    g_start(rows_b, sem_gb, 1)

    def body(p, carry):
        c0 = p * 2
        g_drain(rows_a, sem_ga)
        wa = pltpu.async_copy(
            rows_a, out_hbm.at[pl.ds(base + c0 * _CR, _CR)], sem_wa)
        g_drain(rows_b, sem_gb)
        wb = pltpu.async_copy(
            rows_b, out_hbm.at[pl.ds(base + (c0 + 1) * _CR, _CR)], sem_wb)

        wa.wait()

        @pl.when(p < n_pairs - 1)
        def _():
            g_start(rows_a, sem_ga, c0 + 2)

        wb.wait()

        @pl.when(p < n_pairs - 1)
        def _():
            g_start(rows_b, sem_gb, c0 + 3)

        return carry

    lax.fori_loop(0, n_pairs, body, 0)


def kernel(x, emb_weight):
    n, m = x.shape
    idx2d = x.astype(jnp.int32)
    rpw = n // _NW

    mesh = plsc.VectorSubcoreMesh(core_axis_name="c", subcore_axis_name="s")

    k = functools.partial(
        pl.kernel,
        mesh=mesh,
        out_type=jax.ShapeDtypeStruct((n, m, _D), jnp.float32),
        scratch_types=[
            pltpu.VMEM((rpw, m), jnp.int32),
            pltpu.VMEM((_CR, m, _D), jnp.float32),
            pltpu.VMEM((_CR, m, _D), jnp.float32),
            pltpu.SemaphoreType.DMA,
            pltpu.SemaphoreType.DMA,
            pltpu.SemaphoreType.DMA,
            pltpu.SemaphoreType.DMA,
        ],
        compiler_params=pltpu.CompilerParams(use_tc_tiling_on_sc=False),
    )(functools.partial(_emb_kernel, n, m))

    return k(emb_weight, idx2d)
